# Initial kernel scaffold; baseline (speedup 1.0000x reference)
import jax
import jax.numpy as jnp
from jax.experimental import pallas as pl


def kernel(users, items, user_emb, item_emb, edge_user, edge_item):
    raise NotImplementedError("write your pallas kernel here")


# speedup vs baseline: 7.1815x; 7.1815x over previous
import jax
import jax.numpy as jnp
from jax import lax
from jax.experimental import pallas as pl
from jax.experimental.pallas import tpu as pltpu
from jax.experimental.pallas import tpu_sc as plsc

NU = 25000
NI = 25000
D = 32
NNZ = 800000
BATCH = 16384
K_LAYERS = 3
LAMBDA2 = 5.0
BETA = 0.5

NC = 2
NS = 16
NW = NC * NS

NP = 25600
RT = NP // NS
Q = 160

NNZP = 811008
EW = NNZP // NW
BLK = 768
NBLK = EW // BLK
NSUB = BLK // 128
NG = BLK // 16

E_T1 = NNZP // NS
B_T1 = 3168
N_B1 = E_T1 // B_T1

PB = BATCH // NW

_MESH = plsc.VectorSubcoreMesh(
    core_axis_name="c", subcore_axis_name="s", num_cores=NC, num_subcores=NS
)

F32 = jnp.float32
I32 = jnp.int32


def _iota16():
    return lax.broadcasted_iota(I32, (16,), 0)


def _splat(v):
    return jnp.full((16,), v, I32)


def _fast_rsqrt(v):
    i = lax.bitcast_convert_type(v, I32)
    i = jnp.int32(0x5F3759DF) - lax.shift_right_logical(i, 1)
    y = lax.bitcast_convert_type(i, F32)
    for _ in range(3):
        y = y * (1.5 - 0.5 * v * y * y)
    return y


def _wid():
    return lax.axis_index("c") * NS + lax.axis_index("s")


def _deg_body(eu, ei, w_out, hist_u, hist_i, idxbuf, wbuf, stage):
    cid = lax.axis_index("c")
    sid = lax.axis_index("s")
    z16 = jnp.zeros((16,), F32)
    ones = jnp.ones((16,), F32)

    for hist in (hist_u, hist_i):
        def zero_hist(i, _, hist=hist):
            hist[pl.ds(i * 16, 16)] = z16
            return 0

        lax.fori_loop(0, NP // 16, zero_hist, 0)

    for arr, hist in ((eu, hist_u), (ei, hist_i)):
        def blk(b, _, arr=arr, hist=hist):
            base = pl.multiple_of(sid * E_T1 + b * B_T1, 32)
            pltpu.sync_copy(arr.at[pl.ds(base, B_T1)], idxbuf)

            def step(k, _):
                idxv = idxbuf[pl.ds(k * 16, 16)]
                plsc.addupdate_scatter(hist, (idxv,), ones)
                return 0

            lax.fori_loop(0, B_T1 // 16, step, 0)
            return 0

        lax.fori_loop(0, N_B1, blk, 0)

    for side, hist in ((0, hist_u), (1, hist_i)):
        pltpu.sync_copy(hist, stage.at[pl.ds(pl.multiple_of(sid * NP, 8), NP)])
        plsc.subcore_barrier()

        for r in range(NS):
            pltpu.sync_copy(
                stage.at[pl.ds(pl.multiple_of(r * NP + sid * RT, 8), RT)],
                hist.at[pl.ds(r * RT, RT)])

        def wstep(j, _, hist=hist):
            acc = z16
            for r in range(NS):
                acc = acc + hist[pl.ds(r * RT + j * 16, 16)]
            wv = jnp.where(acc > 0.0, _fast_rsqrt(jnp.maximum(acc, 1.0)), 0.0)
            wbuf[pl.ds(j * 16, 16)] = wv
            return 0

        lax.fori_loop(0, RT // 16, wstep, 0)

        @pl.when(cid == side)
        def _(side=side):
            off = pl.multiple_of(side * NP + sid * RT, 8)
            pltpu.sync_copy(wbuf, w_out.at[pl.ds(off, RT)])

        plsc.subcore_barrier()


def _build_tables(sid, emb_u, emb_i, w, susi, tbuf, wvb, T, mode, sides,
                  dst_off_by_side):
    for side in sides:
        emb = emb_u if side == 0 else emb_i
        for q in range(RT // Q):
            srow = pl.multiple_of(sid * RT + q * Q, 8)
            pltpu.sync_copy(emb.at[pl.ds(srow, Q)], tbuf.at[pl.ds(0, Q)])
            pltpu.sync_copy(
                w.at[pl.ds(pl.multiple_of(side * NP + sid * RT + q * Q, 8),
                           Q)], wvb)
            if susi is not None:
                arow = pl.multiple_of(side * NP + sid * RT + q * Q, 8)
                pltpu.sync_copy(susi.at[0, pl.ds(arow, Q)],
                                tbuf.at[pl.ds(Q, Q)])
                pltpu.sync_copy(susi.at[1, pl.ds(arow, Q)],
                                tbuf.at[pl.ds(2 * Q, Q)])

            def grp(j, _, side=side):
                ridx = j * 16 + _iota16()
                wv = wvb[pl.ds(j * 16, 16)]
                if mode == "PM":
                    wb1 = wv * (BETA if side == 0 else -BETA)
                    wb2 = (wv * wv) * (-BETA)
                else:
                    wb1 = None
                    wb2 = -wv if side == 0 else wv
                for d in range(D):
                    cd = _splat(d)
                    e = plsc.load_gather(tbuf, (ridx, cd))
                    val = e * wb1 if wb1 is not None else e
                    if susi is not None:
                        a = (plsc.load_gather(tbuf, (Q + ridx, cd))
                             + plsc.load_gather(tbuf, (2 * Q + ridx, cd)))
                        val = val + a * wb2
                    plsc.store_scatter(tbuf, (3 * Q + ridx, cd), val)
                return 0

            lax.fori_loop(0, Q // 16, grp, 0)
            drow = pl.multiple_of(
                dst_off_by_side[side] + sid * RT + q * Q, 8)
            pltpu.sync_copy(tbuf.at[pl.ds(3 * Q, Q)], T.at[pl.ds(drow, Q)])


def _edge_update(wid, eu, ei, z_in, z_out, T, iu, ii, tbuf):
    def blk(b, _):
        base = pl.multiple_of(wid * EW + b * BLK, 128)
        pltpu.sync_copy(eu.at[pl.ds(base, BLK)], iu)
        pltpu.sync_copy(ei.at[pl.ds(base, BLK)], ii)

        def off(k, _):
            ii[pl.ds(k * 16, 16)] = ii[pl.ds(k * 16, 16)] + NP
            return 0

        lax.fori_loop(0, BLK // 16, off, 0)
        if z_in is not None:
            pltpu.sync_copy(z_in.at[pl.ds(base, BLK)], tbuf)
        for s in range(NSUB):
            dstv = tbuf.at[pl.ds(s * 128, 128)]
            pltpu.sync_copy(T.at[iu.at[pl.ds(s * 128, 128)]], dstv,
                            add=z_in is not None)
            pltpu.sync_copy(T.at[ii.at[pl.ds(s * 128, 128)]], dstv, add=True)

        def grp(j, _):
            ridx = j * 16 + _iota16()
            sq = jnp.zeros((16,), F32)
            vs = []
            for d in range(D):
                v = plsc.load_gather(tbuf, (ridx, _splat(d)))
                vs.append(v)
                sq = sq + v * v
            scale = jnp.minimum(
                1.0, LAMBDA2 * _fast_rsqrt(jnp.maximum(sq, 1e-24)))
            for d in range(D):
                plsc.store_scatter(tbuf, (ridx, _splat(d)), vs[d] * scale)
            return 0

        lax.fori_loop(0, NG, grp, 0)
        pltpu.sync_copy(tbuf, z_out.at[pl.ds(base, BLK)])
        return 0

    lax.fori_loop(0, NBLK, blk, 0)


def _z1_body(eu, ei, emb_u, emb_i, w, z_out, iu, ii, tbuf, wvb, T):
    sid = lax.axis_index("s")
    _build_tables(sid, emb_u, emb_i, w, None, tbuf, wvb, T, "PM",
                  (0, 1), (0, NP))
    plsc.subcore_barrier()
    _edge_update(_wid(), eu, ei, None, z_out, T, iu, ii, tbuf)


def _zk_body(eu, ei, emb_u, emb_i, w, susi, z_in, z_out,
             iu, ii, tbuf, wvb, T):
    sid = lax.axis_index("s")
    _build_tables(sid, emb_u, emb_i, w, susi, tbuf, wvb, T, "PM",
                  (0, 1), (0, NP))
    plsc.subcore_barrier()
    _edge_update(_wid(), eu, ei, z_in, z_out, T, iu, ii, tbuf)


def _scatter_body(eu3, ei3, z_in, s_out, iu2, ii2, zbuf, S):
    cid = lax.axis_index("c")
    sid = lax.axis_index("s")
    wid = cid * NS + sid
    z16 = jnp.zeros((16,), F32)

    def zrow(r, _):
        plsc.store_scatter(zbuf, (_splat(r), _iota16()), z16)
        plsc.store_scatter(zbuf, (_splat(r), 16 + _iota16()), z16)
        return 0

    lax.fori_loop(0, BLK, zrow, 0)
    for half in range(2):
        trow = pl.multiple_of(half * NP + sid * RT, 8)
        pltpu.sync_copy(zbuf.at[pl.ds(0, BLK)], S.at[pl.ds(trow, BLK)])
        pltpu.sync_copy(zbuf.at[pl.ds(0, BLK)],
                        S.at[pl.ds(trow + BLK, BLK)])
        pltpu.sync_copy(zbuf.at[pl.ds(0, RT - 2 * BLK)],
                        S.at[pl.ds(trow + 2 * BLK, RT - 2 * BLK)])
    plsc.subcore_barrier()

    def blk(b, _):
        base = pl.multiple_of(wid * EW + b * BLK, 128)
        brow = pl.multiple_of(wid * (EW // 128) + b * NSUB, 2)
        pltpu.sync_copy(z_in.at[pl.ds(base, BLK)], zbuf)
        pltpu.sync_copy(eu3.at[pl.ds(brow, NSUB)], iu2)
        pltpu.sync_copy(ei3.at[pl.ds(brow, NSUB)], ii2)
        for r in range(NSUB):
            for k in range(8):
                ii2[r, pl.ds(k * 16, 16)] = ii2[r, pl.ds(k * 16, 16)] + NP
        for s in range(NSUB):
            srcv = zbuf.at[pl.ds(s * 128, 128)]
            pltpu.sync_copy(srcv, S.at[iu2.at[s]], add=True)
            pltpu.sync_copy(srcv, S.at[ii2.at[s]], add=True)
        return 0

    lax.fori_loop(0, NBLK, blk, 0)
    plsc.subcore_barrier()
    for half in range(2):
        trow = pl.multiple_of(half * NP + sid * RT, 8)
        pltpu.sync_copy(S.at[pl.ds(trow, RT)],
                        s_out.at[cid, pl.ds(trow, RT)])


def _final_body(users, items, emb_u, emb_i, w, susi, gamma,
                ub, ibx, rows_u, rows_i, gbuf, tbuf, wvb, T):
    sid = lax.axis_index("s")
    wid = _wid()
    base = pl.multiple_of(wid * PB, 8)
    pltpu.sync_copy(users.at[pl.ds(base, PB)], ub)
    pltpu.sync_copy(items.at[pl.ds(base, PB)], ibx)
    for side, idxr, rows in ((0, ub, rows_u), (1, ibx, rows_i)):
        _build_tables(sid, emb_u, emb_i, w, susi, tbuf, wvb, T, "X",
                      (side,), (0, 0))
        plsc.subcore_barrier()
        for s in range(PB // 128):
            pltpu.sync_copy(T.at[idxr.at[pl.ds(s * 128, 128)]],
                            rows.at[pl.ds(s * 128, 128)])
        plsc.subcore_barrier()

    def grp(j, _):
        ridx = j * 16 + _iota16()
        acc = jnp.zeros((16,), F32)
        for d in range(D):
            cd = _splat(d)
            acc = acc + (plsc.load_gather(rows_u, (ridx, cd))
                         * plsc.load_gather(rows_i, (ridx, cd)))
        gbuf[pl.ds(j * 16, 16)] = acc
        return 0

    lax.fori_loop(0, PB // 16, grp, 0)
    pltpu.sync_copy(gbuf, gamma.at[pl.ds(base, PB)])


def _mk(body, out_type, scratch):
    return pl.kernel(
        body, out_type=out_type, mesh=_MESH, scratch_types=scratch,
        compiler_params=pltpu.CompilerParams(
            needs_layout_passes=False, use_tc_tiling_on_sc=False))


def kernel(users, items, user_emb, item_emb, edge_user, edge_item):
    pad_e = jnp.full((NNZP - NNZ,), NU, I32)
    eu = jnp.concatenate([edge_user.astype(I32), pad_e])
    ei = jnp.concatenate([edge_item.astype(I32), pad_e])
    eu3 = eu.reshape(NNZP // 128, 128)
    ei3 = ei.reshape(NNZP // 128, 128)
    emb_u = jnp.pad(user_emb, ((0, NP - NU), (0, 0)))
    emb_i = jnp.pad(item_emb, ((0, NP - NI), (0, 0)))

    k_deg = _mk(_deg_body, jax.ShapeDtypeStruct((2 * NP,), F32), [
        pltpu.VMEM((NP,), F32),
        pltpu.VMEM((NP,), F32),
        pltpu.VMEM((B_T1,), I32),
        pltpu.VMEM((RT,), F32),
        pltpu.VMEM_SHARED((NS * NP,), F32),
    ])
    w = k_deg(eu, ei)

    zpass_scratch = [
        pltpu.VMEM((BLK,), I32),
        pltpu.VMEM((BLK,), I32),
        pltpu.VMEM((BLK, D), F32),
        pltpu.VMEM((Q,), F32),
        pltpu.VMEM_SHARED((2 * NP, D), F32),
    ]
    zshape = jax.ShapeDtypeStruct((NNZP, D), F32)
    sshape = jax.ShapeDtypeStruct((NC, 2 * NP, D), F32)
    scat_scratch = [
        pltpu.VMEM((NSUB, 128), I32),
        pltpu.VMEM((NSUB, 128), I32),
        pltpu.VMEM((BLK, D), F32),
        pltpu.VMEM_SHARED((2 * NP, D), F32),
    ]

    k_z1 = _mk(_z1_body, zshape, zpass_scratch)
    k_zk = _mk(_zk_body, zshape, zpass_scratch)
    k_sc = _mk(_scatter_body, sshape, scat_scratch)
    k_fin = _mk(_final_body, jax.ShapeDtypeStruct((BATCH,), F32), [
        pltpu.VMEM((PB,), I32),
        pltpu.VMEM((PB,), I32),
        pltpu.VMEM((PB, D), F32),
        pltpu.VMEM((PB, D), F32),
        pltpu.VMEM((PB,), F32),
        pltpu.VMEM((4 * Q, D), F32),
        pltpu.VMEM((Q,), F32),
        pltpu.VMEM_SHARED((NP, D), F32),
    ])

    z = k_z1(eu, ei, emb_u, emb_i, w)
    for _ in range(K_LAYERS - 1):
        s_part = k_sc(eu3, ei3, z)
        z = k_zk(eu, ei, emb_u, emb_i, w, s_part, z)
    s_part = k_sc(eu3, ei3, z)

    gamma = k_fin(users.astype(I32), items.astype(I32), emb_u, emb_i,
                  w, s_part)
    return gamma


# speedup vs baseline: 8.0552x; 1.1217x over previous
import jax
import jax.numpy as jnp
from jax import lax
from jax.experimental import pallas as pl
from jax.experimental.pallas import tpu as pltpu
from jax.experimental.pallas import tpu_sc as plsc

NU = 25000
NI = 25000
D = 32
NNZ = 800000
BATCH = 16384
K_LAYERS = 3
LAMBDA2 = 5.0
BETA = 0.5

NC = 2
NS = 16
NW = NC * NS

NP = 25600
RT = NP // NS
Q = 80

NNZP = 811008
EW = NNZP // NW
BLK = 384
NBLK = EW // BLK
NSUB = BLK // 128
NG = BLK // 16

E_T1 = NNZP // NS
B_T1 = 3168
N_B1 = E_T1 // B_T1

PB = BATCH // NW

_MESH = plsc.VectorSubcoreMesh(
    core_axis_name="c", subcore_axis_name="s", num_cores=NC, num_subcores=NS
)

F32 = jnp.float32
I32 = jnp.int32


def _iota16():
    return lax.broadcasted_iota(I32, (16,), 0)


def _splat(v):
    return jnp.full((16,), v, I32)


def _fast_rsqrt(v):
    i = lax.bitcast_convert_type(v, I32)
    i = jnp.int32(0x5F3759DF) - lax.shift_right_logical(i, 1)
    y = lax.bitcast_convert_type(i, F32)
    for _ in range(3):
        y = y * (1.5 - 0.5 * v * y * y)
    return y


def _wid():
    return lax.axis_index("c") * NS + lax.axis_index("s")


def _deg_body(eu, ei, w_out, hist_u, hist_i, idxbuf, wbuf, stage):
    cid = lax.axis_index("c")
    sid = lax.axis_index("s")
    z16 = jnp.zeros((16,), F32)
    ones = jnp.ones((16,), F32)

    for hist in (hist_u, hist_i):
        def zero_hist(i, _, hist=hist):
            hist[pl.ds(i * 16, 16)] = z16
            return 0

        lax.fori_loop(0, NP // 16, zero_hist, 0)

    for arr, hist in ((eu, hist_u), (ei, hist_i)):
        def blk(b, _, arr=arr, hist=hist):
            base = pl.multiple_of(sid * E_T1 + b * B_T1, 32)
            pltpu.sync_copy(arr.at[pl.ds(base, B_T1)], idxbuf)

            def step(k, _):
                idxv = idxbuf[pl.ds(k * 16, 16)]
                plsc.addupdate_scatter(hist, (idxv,), ones)
                return 0

            lax.fori_loop(0, B_T1 // 16, step, 0)
            return 0

        lax.fori_loop(0, N_B1, blk, 0)

    for side, hist in ((0, hist_u), (1, hist_i)):
        pltpu.sync_copy(hist, stage.at[pl.ds(pl.multiple_of(sid * NP, 8), NP)])
        plsc.subcore_barrier()

        for r in range(NS):
            pltpu.sync_copy(
                stage.at[pl.ds(pl.multiple_of(r * NP + sid * RT, 8), RT)],
                hist.at[pl.ds(r * RT, RT)])

        def wstep(j, _, hist=hist):
            acc = z16
            for r in range(NS):
                acc = acc + hist[pl.ds(r * RT + j * 16, 16)]
            wv = jnp.where(acc > 0.0, _fast_rsqrt(jnp.maximum(acc, 1.0)), 0.0)
            wbuf[pl.ds(j * 16, 16)] = wv
            return 0

        lax.fori_loop(0, RT // 16, wstep, 0)

        @pl.when(cid == side)
        def _(side=side):
            off = pl.multiple_of(side * NP + sid * RT, 8)
            pltpu.sync_copy(wbuf, w_out.at[pl.ds(off, RT)])

        plsc.subcore_barrier()


def _build_tables(sid, emb_u, emb_i, w, susi, tbuf, wvb, T, mode, sides,
                  dst_off_by_side):
    for side in sides:
        emb = emb_u if side == 0 else emb_i

        def qstep(q, _, side=side, emb=emb):
            srow = pl.multiple_of(sid * RT + q * Q, 8)
            pltpu.sync_copy(emb.at[pl.ds(srow, Q)], tbuf.at[pl.ds(0, Q)])
            pltpu.sync_copy(
                w.at[pl.ds(pl.multiple_of(side * NP + sid * RT + q * Q, 8),
                           Q)], wvb)
            if susi is not None:
                arow = pl.multiple_of(side * NP + sid * RT + q * Q, 8)
                pltpu.sync_copy(susi.at[0, pl.ds(arow, Q)],
                                tbuf.at[pl.ds(Q, Q)])
                pltpu.sync_copy(susi.at[1, pl.ds(arow, Q)],
                                tbuf.at[pl.ds(2 * Q, Q)])

            def grp(j, _, side=side):
                ridx = j * 16 + _iota16()
                wv = wvb[pl.ds(j * 16, 16)]
                if mode == "PM":
                    wb1 = wv * (BETA if side == 0 else -BETA)
                    wb2 = (wv * wv) * (-BETA)
                else:
                    wb1 = None
                    wb2 = -wv if side == 0 else wv
                for d in range(D):
                    cd = _splat(d)
                    e = plsc.load_gather(tbuf, (ridx, cd))
                    val = e * wb1 if wb1 is not None else e
                    if susi is not None:
                        a = (plsc.load_gather(tbuf, (Q + ridx, cd))
                             + plsc.load_gather(tbuf, (2 * Q + ridx, cd)))
                        val = val + a * wb2
                    plsc.store_scatter(tbuf, (3 * Q + ridx, cd), val)
                return 0

            lax.fori_loop(0, Q // 16, grp, 0)
            drow = pl.multiple_of(
                dst_off_by_side[side] + sid * RT + q * Q, 8)
            pltpu.sync_copy(tbuf.at[pl.ds(3 * Q, Q)], T.at[pl.ds(drow, Q)])
            return 0

        lax.fori_loop(0, RT // Q, qstep, 0)


def _edge_update(wid, eu, eio, z_in, z_out, T, bufs, ld, gt, st):
    first_pass = z_in is None

    def ebase(b):
        return pl.multiple_of(wid * EW + b * BLK, 128)

    def issue_load(b, s):
        iu, ii, tb = bufs[s]
        base = ebase(b)
        pltpu.async_copy(eu.at[pl.ds(base, BLK)], iu, ld[s])
        pltpu.async_copy(eio.at[pl.ds(base, BLK)], ii, ld[s])
        if not first_pass:
            pltpu.async_copy(z_in.at[pl.ds(base, BLK)], tb, ld[s])

    def wait_load(s):
        iu, ii, tb = bufs[s]
        pltpu.make_async_copy(eu.at[pl.ds(0, BLK)], iu, ld[s]).wait()
        pltpu.make_async_copy(eio.at[pl.ds(0, BLK)], ii, ld[s]).wait()
        if not first_pass:
            pltpu.make_async_copy(z_in.at[pl.ds(0, BLK)], tb, ld[s]).wait()

    def gather_side(s, idxr, add):
        _, _, tb = bufs[s]
        for c in range(NSUB):
            pltpu.async_copy(T.at[idxr.at[pl.ds(c * 128, 128)]],
                             tb.at[pl.ds(c * 128, 128)], gt[s], add=add)

    def wait_gathers(s, n):
        _, _, tb = bufs[s]
        for c in range(n):
            pltpu.make_async_copy(T.at[bufs[s][0].at[pl.ds(0, 128)]],
                                  tb.at[pl.ds((c % NSUB) * 128, 128)],
                                  gt[s]).wait()

    def compute(s):
        _, _, tb = bufs[s]

        def grp(j, _):
            ridx = j * 16 + _iota16()
            sq = jnp.zeros((16,), F32)
            vs = []
            for d in range(D):
                v = plsc.load_gather(tb, (ridx, _splat(d)))
                vs.append(v)
                sq = sq + v * v
            scale = jnp.minimum(
                1.0, LAMBDA2 * _fast_rsqrt(jnp.maximum(sq, 1e-24)))
            for d in range(D):
                plsc.store_scatter(tb, (ridx, _splat(d)), vs[d] * scale)
            return 0

        lax.fori_loop(0, NG, grp, 0)

    def issue_store(b, s):
        _, _, tb = bufs[s]
        pltpu.async_copy(tb, z_out.at[pl.ds(ebase(b), BLK)], st[s])

    def wait_store(s):
        _, _, tb = bufs[s]
        pltpu.make_async_copy(tb, z_out.at[pl.ds(0, BLK)], st[s]).wait()

    def body(b, s, o, prefetch, first_waits):
        wait_load(s)
        if first_pass:
            gather_side(s, bufs[s][0], False)
            wait_gathers(s, NSUB)
            gather_side(s, bufs[s][1], True)
            wait_gathers(s, NSUB)
        else:
            gather_side(s, bufs[s][0], True)
            gather_side(s, bufs[s][1], True)
            wait_gathers(s, 2 * NSUB)
        if prefetch:
            if not first_waits:
                wait_store(o)
            issue_load(b + 1, o)
        compute(s)
        issue_store(b, s)

    issue_load(0, 0)
    body(0, 0, 1, True, True)
    body(1, 1, 0, True, False)

    def pair(p, _):
        b = 2 * p + 2
        body(b, 0, 1, True, False)
        body(b + 1, 1, 0, True, False)
        return 0

    lax.fori_loop(0, (NBLK - 4) // 2, pair, 0)
    body(NBLK - 2, 0, 1, True, False)
    body(NBLK - 1, 1, 0, False, False)
    wait_store(0)
    wait_store(1)


def _z1_body(eu, eio, emb_u, emb_i, w, z_out,
             iu0, ii0, tb0, iu1, ii1, tb1, wvb, T,
             ld0, ld1, gt0, gt1, st0, st1):
    sid = lax.axis_index("s")
    _build_tables(sid, emb_u, emb_i, w, None, tb0, wvb, T, "PM",
                  (0, 1), (0, NP))
    plsc.subcore_barrier()
    _edge_update(_wid(), eu, eio, None, z_out, T,
                 ((iu0, ii0, tb0), (iu1, ii1, tb1)),
                 (ld0, ld1), (gt0, gt1), (st0, st1))


def _zk_body(eu, eio, emb_u, emb_i, w, susi, z_in, z_out,
             iu0, ii0, tb0, iu1, ii1, tb1, wvb, T,
             ld0, ld1, gt0, gt1, st0, st1):
    sid = lax.axis_index("s")
    _build_tables(sid, emb_u, emb_i, w, susi, tb0, wvb, T, "PM",
                  (0, 1), (0, NP))
    plsc.subcore_barrier()
    _edge_update(_wid(), eu, eio, z_in, z_out, T,
                 ((iu0, ii0, tb0), (iu1, ii1, tb1)),
                 (ld0, ld1), (gt0, gt1), (st0, st1))


def _scatter_body(eu, eio, z_in, s_out,
                  iu20, ii20, zb0, iu21, ii21, zb1,
                  S, ld0, ld1, sc0, sc1):
    cid = lax.axis_index("c")
    sid = lax.axis_index("s")
    wid = cid * NS + sid
    z16 = jnp.zeros((16,), F32)
    bufs = ((iu20, ii20, zb0), (iu21, ii21, zb1))
    ld = (ld0, ld1)
    sc = (sc0, sc1)

    def zrow(r, _):
        plsc.store_scatter(zb0, (_splat(r), _iota16()), z16)
        plsc.store_scatter(zb0, (_splat(r), 16 + _iota16()), z16)
        return 0

    lax.fori_loop(0, BLK, zrow, 0)
    for half in range(2):
        trow = pl.multiple_of(half * NP + sid * RT, 8)
        for c0 in range(0, RT, BLK):
            n = min(BLK, RT - c0)
            pltpu.sync_copy(zb0.at[pl.ds(0, n)], S.at[pl.ds(trow + c0, n)])
    plsc.subcore_barrier()

    def issue_load(b, s):
        iu2, ii2, zb = bufs[s]
        base = pl.multiple_of(wid * EW + b * BLK, 128)
        pltpu.async_copy(z_in.at[pl.ds(base, BLK)], zb, ld[s])
        pltpu.async_copy(eu.at[pl.ds(base, BLK)], iu2, ld[s])
        pltpu.async_copy(eio.at[pl.ds(base, BLK)], ii2, ld[s])

    def wait_load(s):
        iu2, ii2, zb = bufs[s]
        pltpu.make_async_copy(z_in.at[pl.ds(0, BLK)], zb, ld[s]).wait()
        pltpu.make_async_copy(eu.at[pl.ds(0, BLK)], iu2, ld[s]).wait()
        pltpu.make_async_copy(eio.at[pl.ds(0, BLK)], ii2, ld[s]).wait()

    def issue_scatters(s):
        iu2, ii2, zb = bufs[s]
        for c in range(NSUB):
            srcv = zb.at[pl.ds(c * 128, 128)]
            pltpu.async_copy(srcv, S.at[iu2.at[pl.ds(c * 128, 128)]],
                             sc[s], add=True)
            pltpu.async_copy(srcv, S.at[ii2.at[pl.ds(c * 128, 128)]],
                             sc[s], add=True)

    def wait_scatters(s):
        iu2, ii2, zb = bufs[s]
        for c in range(NSUB):
            pltpu.make_async_copy(zb.at[pl.ds(c * 128, 128)],
                                  S.at[iu2.at[pl.ds(c * 128, 128)]],
                                  sc[s]).wait()
            pltpu.make_async_copy(zb.at[pl.ds(c * 128, 128)],
                                  S.at[ii2.at[pl.ds(c * 128, 128)]],
                                  sc[s]).wait()

    def body(b, s, o, prefetch, first):
        wait_load(s)
        issue_scatters(s)
        if prefetch:
            if not first:
                wait_scatters(o)
            issue_load(b + 1, o)

    issue_load(0, 0)
    body(0, 0, 1, True, True)
    body(1, 1, 0, True, False)

    def pair(p, _):
        b = 2 * p + 2
        body(b, 0, 1, True, False)
        body(b + 1, 1, 0, True, False)
        return 0

    lax.fori_loop(0, (NBLK - 4) // 2, pair, 0)
    body(NBLK - 2, 0, 1, True, False)
    body(NBLK - 1, 1, 0, False, False)
    wait_scatters(0)
    wait_scatters(1)

    plsc.subcore_barrier()
    for half in range(2):
        trow = pl.multiple_of(half * NP + sid * RT, 8)
        pltpu.sync_copy(S.at[pl.ds(trow, RT)],
                        s_out.at[cid, pl.ds(trow, RT)])


def _final_body(users, items, emb_u, emb_i, w, susi, gamma,
                ub, ibx, rows_u, rows_i, gbuf, tbuf, wvb, T):
    sid = lax.axis_index("s")
    wid = _wid()
    base = pl.multiple_of(wid * PB, 8)
    pltpu.sync_copy(users.at[pl.ds(base, PB)], ub)
    pltpu.sync_copy(items.at[pl.ds(base, PB)], ibx)
    for side, idxr, rows in ((0, ub, rows_u), (1, ibx, rows_i)):
        _build_tables(sid, emb_u, emb_i, w, susi, tbuf, wvb, T, "X",
                      (side,), (0, 0))
        plsc.subcore_barrier()
        for s in range(PB // 128):
            pltpu.sync_copy(T.at[idxr.at[pl.ds(s * 128, 128)]],
                            rows.at[pl.ds(s * 128, 128)])
        plsc.subcore_barrier()

    def grp(j, _):
        ridx = j * 16 + _iota16()
        acc = jnp.zeros((16,), F32)
        for d in range(D):
            cd = _splat(d)
            acc = acc + (plsc.load_gather(rows_u, (ridx, cd))
                         * plsc.load_gather(rows_i, (ridx, cd)))
        gbuf[pl.ds(j * 16, 16)] = acc
        return 0

    lax.fori_loop(0, PB // 16, grp, 0)
    pltpu.sync_copy(gbuf, gamma.at[pl.ds(base, PB)])


def _mk(body, out_type, scratch):
    return pl.kernel(
        body, out_type=out_type, mesh=_MESH, scratch_types=scratch,
        compiler_params=pltpu.CompilerParams(
            needs_layout_passes=False, use_tc_tiling_on_sc=False))


def kernel(users, items, user_emb, item_emb, edge_user, edge_item):
    pad_e = jnp.full((NNZP - NNZ,), NU, I32)
    eu = jnp.concatenate([edge_user.astype(I32), pad_e])
    ei = jnp.concatenate([edge_item.astype(I32), pad_e])
    eio = ei + NP
    emb_u = jnp.pad(user_emb, ((0, NP - NU), (0, 0)))
    emb_i = jnp.pad(item_emb, ((0, NP - NI), (0, 0)))

    k_deg = _mk(_deg_body, jax.ShapeDtypeStruct((2 * NP,), F32), [
        pltpu.VMEM((NP,), F32),
        pltpu.VMEM((NP,), F32),
        pltpu.VMEM((B_T1,), I32),
        pltpu.VMEM((RT,), F32),
        pltpu.VMEM_SHARED((NS * NP,), F32),
    ])
    w = k_deg(eu, ei)

    sems6 = [pltpu.SemaphoreType.DMA] * 6
    sems4 = [pltpu.SemaphoreType.DMA] * 4
    slot = [
        pltpu.VMEM((BLK,), I32),
        pltpu.VMEM((BLK,), I32),
        pltpu.VMEM((BLK, D), F32),
    ]
    zpass_scratch = (slot + slot + [
        pltpu.VMEM((Q,), F32),
        pltpu.VMEM_SHARED((2 * NP, D), F32),
    ] + sems6)
    zshape = jax.ShapeDtypeStruct((NNZP, D), F32)
    sshape = jax.ShapeDtypeStruct((NC, 2 * NP, D), F32)
    scat_scratch = (slot + slot + [
        pltpu.VMEM_SHARED((2 * NP, D), F32),
    ] + sems4)

    k_z1 = _mk(_z1_body, zshape, zpass_scratch)
    k_zk = _mk(_zk_body, zshape, zpass_scratch)
    k_sc = _mk(_scatter_body, sshape, scat_scratch)
    k_fin = _mk(_final_body, jax.ShapeDtypeStruct((BATCH,), F32), [
        pltpu.VMEM((PB,), I32),
        pltpu.VMEM((PB,), I32),
        pltpu.VMEM((PB, D), F32),
        pltpu.VMEM((PB, D), F32),
        pltpu.VMEM((PB,), F32),
        pltpu.VMEM((4 * Q, D), F32),
        pltpu.VMEM((Q,), F32),
        pltpu.VMEM_SHARED((NP, D), F32),
    ])

    z = k_z1(eu, eio, emb_u, emb_i, w)
    for _ in range(K_LAYERS - 1):
        s_part = k_sc(eu, eio, z)
        z = k_zk(eu, eio, emb_u, emb_i, w, s_part, z)
    s_part = k_sc(eu, eio, z)

    gamma = k_fin(users.astype(I32), items.astype(I32), emb_u, emb_i,
                  w, s_part)
    return gamma


# speedup vs baseline: 18.1574x; 2.2541x over previous
import jax
import jax.numpy as jnp
from jax import lax
from jax.experimental import pallas as pl
from jax.experimental.pallas import tpu as pltpu
from jax.experimental.pallas import tpu_sc as plsc

NU = 25000
NI = 25000
D = 32
NNZ = 800000
BATCH = 16384
K_LAYERS = 3
LAMBDA2 = 5.0
BETA = 0.5

NC = 2
NS = 16
NW = NC * NS

NP = 25600
RT = NP // NS
Q = 80

NNZP = 811008
EW = NNZP // NW
BLK = 384
NBLK = EW // BLK
NSUB = BLK // 128
NG = BLK // 16

E_T1 = NNZP // NS
B_T1 = 3168
N_B1 = E_T1 // B_T1

PB = BATCH // NW

_MESH = plsc.VectorSubcoreMesh(
    core_axis_name="c", subcore_axis_name="s", num_cores=NC, num_subcores=NS
)

F32 = jnp.float32
I32 = jnp.int32


def _iota16():
    return lax.broadcasted_iota(I32, (16,), 0)


def _splat(v):
    return jnp.full((16,), v, I32)


def _rot_cols():
    it = _iota16()
    return [jnp.bitwise_and(d + it, D - 1) for d in range(D)]


def _fast_rsqrt(v):
    i = lax.bitcast_convert_type(v, I32)
    i = jnp.int32(0x5F3759DF) - lax.shift_right_logical(i, 1)
    y = lax.bitcast_convert_type(i, F32)
    for _ in range(3):
        y = y * (1.5 - 0.5 * v * y * y)
    return y


def _wid():
    return lax.axis_index("c") * NS + lax.axis_index("s")


def _deg_body(eu, ei, w_out, hist_u, hist_i, idxbuf, wbuf, stage):
    cid = lax.axis_index("c")
    sid = lax.axis_index("s")
    z16 = jnp.zeros((16,), F32)
    ones = jnp.ones((16,), F32)

    for hist in (hist_u, hist_i):
        def zero_hist(i, _, hist=hist):
            hist[pl.ds(i * 16, 16)] = z16
            return 0

        lax.fori_loop(0, NP // 16, zero_hist, 0)

    for arr, hist in ((eu, hist_u), (ei, hist_i)):
        def blk(b, _, arr=arr, hist=hist):
            base = pl.multiple_of(sid * E_T1 + b * B_T1, 32)
            pltpu.sync_copy(arr.at[pl.ds(base, B_T1)], idxbuf)

            def step(k, _):
                idxv = idxbuf[pl.ds(k * 16, 16)]
                plsc.addupdate_scatter(hist, (idxv,), ones)
                return 0

            lax.fori_loop(0, B_T1 // 16, step, 0)
            return 0

        lax.fori_loop(0, N_B1, blk, 0)

    for side, hist in ((0, hist_u), (1, hist_i)):
        pltpu.sync_copy(hist, stage.at[pl.ds(pl.multiple_of(sid * NP, 8), NP)])
        plsc.subcore_barrier()

        for r in range(NS):
            pltpu.sync_copy(
                stage.at[pl.ds(pl.multiple_of(r * NP + sid * RT, 8), RT)],
                hist.at[pl.ds(r * RT, RT)])

        def wstep(j, _, hist=hist):
            acc = z16
            for r in range(NS):
                acc = acc + hist[pl.ds(r * RT + j * 16, 16)]
            wv = jnp.where(acc > 0.0, _fast_rsqrt(jnp.maximum(acc, 1.0)), 0.0)
            wbuf[pl.ds(j * 16, 16)] = wv
            return 0

        lax.fori_loop(0, RT // 16, wstep, 0)

        @pl.when(cid == side)
        def _(side=side):
            off = pl.multiple_of(side * NP + sid * RT, 8)
            pltpu.sync_copy(wbuf, w_out.at[pl.ds(off, RT)])

        plsc.subcore_barrier()


def _build_tables(sid, emb_u, emb_i, w, susi, tbuf, wvb, T, mode, sides,
                  dst_off_by_side):
    for side in sides:
        emb = emb_u if side == 0 else emb_i

        def qstep(q, _, side=side, emb=emb):
            srow = pl.multiple_of(sid * RT + q * Q, 8)
            pltpu.sync_copy(emb.at[pl.ds(srow, Q)], tbuf.at[pl.ds(0, Q)])
            pltpu.sync_copy(
                w.at[pl.ds(pl.multiple_of(side * NP + sid * RT + q * Q, 8),
                           Q)], wvb)
            if susi is not None:
                arow = pl.multiple_of(side * NP + sid * RT + q * Q, 8)
                pltpu.sync_copy(susi.at[0, pl.ds(arow, Q)],
                                tbuf.at[pl.ds(Q, Q)])
                pltpu.sync_copy(susi.at[1, pl.ds(arow, Q)],
                                tbuf.at[pl.ds(2 * Q, Q)])

            def grp(j, _, side=side):
                ridx = j * 16 + _iota16()
                cds = _rot_cols()
                wv = wvb[pl.ds(j * 16, 16)]
                if mode == "PM":
                    wb1 = wv * (BETA if side == 0 else -BETA)
                    wb2 = (wv * wv) * (-BETA)
                else:
                    wb1 = None
                    wb2 = -wv if side == 0 else wv
                for d in range(D):
                    cd = cds[d]
                    e = plsc.load_gather(tbuf, (ridx, cd))
                    val = e * wb1 if wb1 is not None else e
                    if susi is not None:
                        a = (plsc.load_gather(tbuf, (Q + ridx, cd))
                             + plsc.load_gather(tbuf, (2 * Q + ridx, cd)))
                        val = val + a * wb2
                    plsc.store_scatter(tbuf, (3 * Q + ridx, cd), val)
                return 0

            lax.fori_loop(0, Q // 16, grp, 0)
            drow = pl.multiple_of(
                dst_off_by_side[side] + sid * RT + q * Q, 8)
            pltpu.sync_copy(tbuf.at[pl.ds(3 * Q, Q)], T.at[pl.ds(drow, Q)])
            return 0

        lax.fori_loop(0, RT // Q, qstep, 0)


def _edge_update(wid, eu, eio, z_in, z_out, T, bufs, ld, gt, st):
    first_pass = z_in is None

    def ebase(b):
        return pl.multiple_of(wid * EW + b * BLK, 128)

    def issue_load(b, s):
        iu, ii, tb = bufs[s]
        base = ebase(b)
        pltpu.async_copy(eu.at[pl.ds(base, BLK)], iu, ld[s])
        pltpu.async_copy(eio.at[pl.ds(base, BLK)], ii, ld[s])
        if not first_pass:
            pltpu.async_copy(z_in.at[pl.ds(base, BLK)], tb, ld[s])

    def wait_load(s):
        iu, ii, tb = bufs[s]
        pltpu.make_async_copy(eu.at[pl.ds(0, BLK)], iu, ld[s]).wait()
        pltpu.make_async_copy(eio.at[pl.ds(0, BLK)], ii, ld[s]).wait()
        if not first_pass:
            pltpu.make_async_copy(z_in.at[pl.ds(0, BLK)], tb, ld[s]).wait()

    def gather_side(s, idxr, add):
        _, _, tb = bufs[s]
        for c in range(NSUB):
            pltpu.async_copy(T.at[idxr.at[pl.ds(c * 128, 128)]],
                             tb.at[pl.ds(c * 128, 128)], gt[s], add=add)

    def wait_gathers(s, n):
        _, _, tb = bufs[s]
        for c in range(n):
            pltpu.make_async_copy(T.at[bufs[s][0].at[pl.ds(0, 128)]],
                                  tb.at[pl.ds((c % NSUB) * 128, 128)],
                                  gt[s]).wait()

    def compute(s):
        _, _, tb = bufs[s]

        def grp(j, _):
            ridx = j * 16 + _iota16()
            cds = _rot_cols()
            parts = [jnp.zeros((16,), F32) for _ in range(4)]
            vs = []
            for d in range(D):
                v = plsc.load_gather(tb, (ridx, cds[d]))
                vs.append(v)
                parts[d % 4] = parts[d % 4] + v * v
            sq = (parts[0] + parts[1]) + (parts[2] + parts[3])
            scale = jnp.minimum(
                1.0, LAMBDA2 * _fast_rsqrt(jnp.maximum(sq, 1e-24)))
            for d in range(D):
                plsc.store_scatter(tb, (ridx, cds[d]), vs[d] * scale)
            return 0

        lax.fori_loop(0, NG, grp, 0)

    def issue_store(b, s):
        _, _, tb = bufs[s]
        pltpu.async_copy(tb, z_out.at[pl.ds(ebase(b), BLK)], st[s])

    def wait_store(s):
        _, _, tb = bufs[s]
        pltpu.make_async_copy(tb, z_out.at[pl.ds(0, BLK)], st[s]).wait()

    def body(b, s, o, prefetch, first_waits):
        wait_load(s)
        if first_pass:
            gather_side(s, bufs[s][0], False)
            wait_gathers(s, NSUB)
            gather_side(s, bufs[s][1], True)
            wait_gathers(s, NSUB)
        else:
            gather_side(s, bufs[s][0], True)
            gather_side(s, bufs[s][1], True)
            wait_gathers(s, 2 * NSUB)
        if prefetch:
            if not first_waits:
                wait_store(o)
            issue_load(b + 1, o)
        compute(s)
        issue_store(b, s)

    issue_load(0, 0)
    body(0, 0, 1, True, True)
    body(1, 1, 0, True, False)

    def pair(p, _):
        b = 2 * p + 2
        body(b, 0, 1, True, False)
        body(b + 1, 1, 0, True, False)
        return 0

    lax.fori_loop(0, (NBLK - 4) // 2, pair, 0)
    body(NBLK - 2, 0, 1, True, False)
    body(NBLK - 1, 1, 0, False, False)
    wait_store(0)
    wait_store(1)


def _z1_body(eu, eio, emb_u, emb_i, w, z_out,
             iu0, ii0, tb0, iu1, ii1, tb1, wvb, T,
             ld0, ld1, gt0, gt1, st0, st1):
    sid = lax.axis_index("s")
    _build_tables(sid, emb_u, emb_i, w, None, tb0, wvb, T, "PM",
                  (0, 1), (0, NP))
    plsc.subcore_barrier()
    _edge_update(_wid(), eu, eio, None, z_out, T,
                 ((iu0, ii0, tb0), (iu1, ii1, tb1)),
                 (ld0, ld1), (gt0, gt1), (st0, st1))


def _zk_body(eu, eio, emb_u, emb_i, w, susi, z_in, z_out,
             iu0, ii0, tb0, iu1, ii1, tb1, wvb, T,
             ld0, ld1, gt0, gt1, st0, st1):
    sid = lax.axis_index("s")
    _build_tables(sid, emb_u, emb_i, w, susi, tb0, wvb, T, "PM",
                  (0, 1), (0, NP))
    plsc.subcore_barrier()
    _edge_update(_wid(), eu, eio, z_in, z_out, T,
                 ((iu0, ii0, tb0), (iu1, ii1, tb1)),
                 (ld0, ld1), (gt0, gt1), (st0, st1))


def _scatter_body(eu, eio, z_in, s_out,
                  iu20, ii20, zb0, iu21, ii21, zb1,
                  S, ld0, ld1, sc0, sc1):
    cid = lax.axis_index("c")
    sid = lax.axis_index("s")
    wid = cid * NS + sid
    z16 = jnp.zeros((16,), F32)
    bufs = ((iu20, ii20, zb0), (iu21, ii21, zb1))
    ld = (ld0, ld1)
    sc = (sc0, sc1)

    def zrow(r, _):
        plsc.store_scatter(zb0, (_splat(r), _iota16()), z16)
        plsc.store_scatter(zb0, (_splat(r), 16 + _iota16()), z16)
        return 0

    lax.fori_loop(0, BLK, zrow, 0)
    for half in range(2):
        trow = pl.multiple_of(half * NP + sid * RT, 8)
        for c0 in range(0, RT, BLK):
            n = min(BLK, RT - c0)
            pltpu.sync_copy(zb0.at[pl.ds(0, n)], S.at[pl.ds(trow + c0, n)])
    plsc.subcore_barrier()

    def issue_load(b, s):
        iu2, ii2, zb = bufs[s]
        base = pl.multiple_of(wid * EW + b * BLK, 128)
        pltpu.async_copy(z_in.at[pl.ds(base, BLK)], zb, ld[s])
        pltpu.async_copy(eu.at[pl.ds(base, BLK)], iu2, ld[s])
        pltpu.async_copy(eio.at[pl.ds(base, BLK)], ii2, ld[s])

    def wait_load(s):
        iu2, ii2, zb = bufs[s]
        pltpu.make_async_copy(z_in.at[pl.ds(0, BLK)], zb, ld[s]).wait()
        pltpu.make_async_copy(eu.at[pl.ds(0, BLK)], iu2, ld[s]).wait()
        pltpu.make_async_copy(eio.at[pl.ds(0, BLK)], ii2, ld[s]).wait()

    def issue_scatters(s):
        iu2, ii2, zb = bufs[s]
        for c in range(NSUB):
            srcv = zb.at[pl.ds(c * 128, 128)]
            pltpu.async_copy(srcv, S.at[iu2.at[pl.ds(c * 128, 128)]],
                             sc[s], add=True)
            pltpu.async_copy(srcv, S.at[ii2.at[pl.ds(c * 128, 128)]],
                             sc[s], add=True)

    def wait_scatters(s):
        iu2, ii2, zb = bufs[s]
        for c in range(NSUB):
            pltpu.make_async_copy(zb.at[pl.ds(c * 128, 128)],
                                  S.at[iu2.at[pl.ds(c * 128, 128)]],
                                  sc[s]).wait()
            pltpu.make_async_copy(zb.at[pl.ds(c * 128, 128)],
                                  S.at[ii2.at[pl.ds(c * 128, 128)]],
                                  sc[s]).wait()

    def body(b, s, o, prefetch, first):
        wait_load(s)
        issue_scatters(s)
        if prefetch:
            if not first:
                wait_scatters(o)
            issue_load(b + 1, o)

    issue_load(0, 0)
    body(0, 0, 1, True, True)
    body(1, 1, 0, True, False)

    def pair(p, _):
        b = 2 * p + 2
        body(b, 0, 1, True, False)
        body(b + 1, 1, 0, True, False)
        return 0

    lax.fori_loop(0, (NBLK - 4) // 2, pair, 0)
    body(NBLK - 2, 0, 1, True, False)
    body(NBLK - 1, 1, 0, False, False)
    wait_scatters(0)
    wait_scatters(1)

    plsc.subcore_barrier()
    for half in range(2):
        trow = pl.multiple_of(half * NP + sid * RT, 8)
        pltpu.sync_copy(S.at[pl.ds(trow, RT)],
                        s_out.at[cid, pl.ds(trow, RT)])


def _final_body(users, items, emb_u, emb_i, w, susi, gamma,
                ub, ibx, rows_u, rows_i, gbuf, tbuf, wvb, T):
    sid = lax.axis_index("s")
    wid = _wid()
    base = pl.multiple_of(wid * PB, 8)
    pltpu.sync_copy(users.at[pl.ds(base, PB)], ub)
    pltpu.sync_copy(items.at[pl.ds(base, PB)], ibx)
    for side, idxr, rows in ((0, ub, rows_u), (1, ibx, rows_i)):
        _build_tables(sid, emb_u, emb_i, w, susi, tbuf, wvb, T, "X",
                      (side,), (0, 0))
        plsc.subcore_barrier()
        for s in range(PB // 128):
            pltpu.sync_copy(T.at[idxr.at[pl.ds(s * 128, 128)]],
                            rows.at[pl.ds(s * 128, 128)])
        plsc.subcore_barrier()

    def grp(j, _):
        ridx = j * 16 + _iota16()
        cds = _rot_cols()
        parts = [jnp.zeros((16,), F32) for _ in range(4)]
        for d in range(D):
            cd = cds[d]
            parts[d % 4] = parts[d % 4] + (
                plsc.load_gather(rows_u, (ridx, cd))
                * plsc.load_gather(rows_i, (ridx, cd)))
        gbuf[pl.ds(j * 16, 16)] = (parts[0] + parts[1]) + (parts[2]
                                                           + parts[3])
        return 0

    lax.fori_loop(0, PB // 16, grp, 0)
    pltpu.sync_copy(gbuf, gamma.at[pl.ds(base, PB)])


def _mk(body, out_type, scratch):
    return pl.kernel(
        body, out_type=out_type, mesh=_MESH, scratch_types=scratch,
        compiler_params=pltpu.CompilerParams(
            needs_layout_passes=False, use_tc_tiling_on_sc=False))


def kernel(users, items, user_emb, item_emb, edge_user, edge_item):
    pad_e = jnp.full((NNZP - NNZ,), NU, I32)
    eu = jnp.concatenate([edge_user.astype(I32), pad_e])
    ei = jnp.concatenate([edge_item.astype(I32), pad_e])
    eio = ei + NP
    emb_u = jnp.pad(user_emb, ((0, NP - NU), (0, 0)))
    emb_i = jnp.pad(item_emb, ((0, NP - NI), (0, 0)))

    k_deg = _mk(_deg_body, jax.ShapeDtypeStruct((2 * NP,), F32), [
        pltpu.VMEM((NP,), F32),
        pltpu.VMEM((NP,), F32),
        pltpu.VMEM((B_T1,), I32),
        pltpu.VMEM((RT,), F32),
        pltpu.VMEM_SHARED((NS * NP,), F32),
    ])
    w = k_deg(eu, ei)

    sems6 = [pltpu.SemaphoreType.DMA] * 6
    sems4 = [pltpu.SemaphoreType.DMA] * 4
    slot = [
        pltpu.VMEM((BLK,), I32),
        pltpu.VMEM((BLK,), I32),
        pltpu.VMEM((BLK, D), F32),
    ]
    zpass_scratch = (slot + slot + [
        pltpu.VMEM((Q,), F32),
        pltpu.VMEM_SHARED((2 * NP, D), F32),
    ] + sems6)
    zshape = jax.ShapeDtypeStruct((NNZP, D), F32)
    sshape = jax.ShapeDtypeStruct((NC, 2 * NP, D), F32)
    scat_scratch = (slot + slot + [
        pltpu.VMEM_SHARED((2 * NP, D), F32),
    ] + sems4)

    k_z1 = _mk(_z1_body, zshape, zpass_scratch)
    k_zk = _mk(_zk_body, zshape, zpass_scratch)
    k_sc = _mk(_scatter_body, sshape, scat_scratch)
    k_fin = _mk(_final_body, jax.ShapeDtypeStruct((BATCH,), F32), [
        pltpu.VMEM((PB,), I32),
        pltpu.VMEM((PB,), I32),
        pltpu.VMEM((PB, D), F32),
        pltpu.VMEM((PB, D), F32),
        pltpu.VMEM((PB,), F32),
        pltpu.VMEM((4 * Q, D), F32),
        pltpu.VMEM((Q,), F32),
        pltpu.VMEM_SHARED((NP, D), F32),
    ])

    z = k_z1(eu, eio, emb_u, emb_i, w)
    for _ in range(K_LAYERS - 1):
        s_part = k_sc(eu, eio, z)
        z = k_zk(eu, eio, emb_u, emb_i, w, s_part, z)
    s_part = k_sc(eu, eio, z)

    gamma = k_fin(users.astype(I32), items.astype(I32), emb_u, emb_i,
                  w, s_part)
    return gamma


# speedup vs baseline: 20.6788x; 1.1389x over previous
import jax
import jax.numpy as jnp
from jax import lax
from jax.experimental import pallas as pl
from jax.experimental.pallas import tpu as pltpu
from jax.experimental.pallas import tpu_sc as plsc

NU = 25000
NI = 25000
D = 32
NNZ = 800000
BATCH = 16384
K_LAYERS = 3
LAMBDA2 = 5.0
BETA = 0.5

NC = 2
NS = 16
NW = NC * NS

NP = 25600
RT = NP // NS
Q = 80

NNZP = 811008
EW = NNZP // NW
BLK = 384
NBLK = EW // BLK
NSUB = BLK // 128
NG = BLK // 16

E_T1 = NNZP // NS
B_T1 = 3168
N_B1 = E_T1 // B_T1

PB = BATCH // NW

_MESH = plsc.VectorSubcoreMesh(
    core_axis_name="c", subcore_axis_name="s", num_cores=NC, num_subcores=NS
)

F32 = jnp.float32
I32 = jnp.int32


def _iota16():
    return lax.broadcasted_iota(I32, (16,), 0)


def _splat(v):
    return jnp.full((16,), v, I32)


def _rot_cols():
    it = _iota16()
    return [jnp.bitwise_and(d + it, D - 1) for d in range(D)]


def _fast_rsqrt(v):
    i = lax.bitcast_convert_type(v, I32)
    i = jnp.int32(0x5F3759DF) - lax.shift_right_logical(i, 1)
    y = lax.bitcast_convert_type(i, F32)
    for _ in range(3):
        y = y * (1.5 - 0.5 * v * y * y)
    return y


def _wid():
    return lax.axis_index("c") * NS + lax.axis_index("s")


def _deg_body(eu, ei, w_out, hist_u, hist_i, idxbuf, idxbuf2, wbuf, stage):
    cid = lax.axis_index("c")
    sid = lax.axis_index("s")
    z16 = jnp.zeros((16,), F32)
    ones = jnp.ones((16,), F32)

    for hist in (hist_u, hist_i):
        def zero_hist(i, _, hist=hist):
            hist[pl.ds(i * 16, 16)] = z16
            return 0

        lax.fori_loop(0, NP // 16, zero_hist, 0)

    sched = ([(eu, hist_u, b) for b in range(N_B1)]
             + [(ei, hist_i, b) for b in range(N_B1)])
    ibufs = (idxbuf, idxbuf2)

    def run_hist(dsem):
        def issue(i):
            arr, _, b = sched[i]
            base = pl.multiple_of(sid * E_T1 + b * B_T1, 32)
            pltpu.async_copy(arr.at[pl.ds(base, B_T1)], ibufs[i % 2], dsem)

        issue(0)
        for i in range(len(sched)):
            arr, hist, b = sched[i]
            buf = ibufs[i % 2]
            pltpu.make_async_copy(arr.at[pl.ds(0, B_T1)], buf, dsem).wait()
            if i + 1 < len(sched):
                issue(i + 1)

            def step(k, _, hist=hist, buf=buf):
                idxv = buf[pl.ds(k * 16, 16)]
                plsc.addupdate_scatter(hist, (idxv,), ones)
                return 0

            lax.fori_loop(0, B_T1 // 16, step, 0)

    pl.run_scoped(run_hist, pltpu.SemaphoreType.DMA)

    for side, hist in ((0, hist_u), (1, hist_i)):
        pltpu.sync_copy(hist, stage.at[pl.ds(pl.multiple_of(sid * NP, 8), NP)])
        plsc.subcore_barrier()

        pltpu.sync_copy(
            [stage.at[pl.ds(pl.multiple_of(r * NP + sid * RT, 8), RT)]
             for r in range(NS)],
            [hist.at[pl.ds(r * RT, RT)] for r in range(NS)])

        def wstep(j, _, hist=hist):
            acc = z16
            for r in range(NS):
                acc = acc + hist[pl.ds(r * RT + j * 16, 16)]
            wv = jnp.where(acc > 0.0, _fast_rsqrt(jnp.maximum(acc, 1.0)), 0.0)
            wbuf[pl.ds(j * 16, 16)] = wv
            return 0

        lax.fori_loop(0, RT // 16, wstep, 0)

        @pl.when(cid == side)
        def _(side=side):
            off = pl.multiple_of(side * NP + sid * RT, 8)
            pltpu.sync_copy(wbuf, w_out.at[pl.ds(off, RT)])

        plsc.subcore_barrier()


def _build_tables(sid, emb_u, emb_i, w, susi, tbuf, wvb, T, mode, sides,
                  dst_off_by_side):
    for side in sides:
        emb = emb_u if side == 0 else emb_i

        def qstep(q, _, side=side, emb=emb):
            srow = pl.multiple_of(sid * RT + q * Q, 8)
            arow = pl.multiple_of(side * NP + sid * RT + q * Q, 8)
            srcs = [emb.at[pl.ds(srow, Q)], w.at[pl.ds(arow, Q)]]
            dsts = [tbuf.at[pl.ds(0, Q)], wvb]
            if susi is not None:
                srcs += [susi.at[0, pl.ds(arow, Q)],
                         susi.at[1, pl.ds(arow, Q)]]
                dsts += [tbuf.at[pl.ds(Q, Q)], tbuf.at[pl.ds(2 * Q, Q)]]
            pltpu.sync_copy(srcs, dsts)

            def grp(j, _, side=side):
                ridx = j * 16 + _iota16()
                cds = _rot_cols()
                wv = wvb[pl.ds(j * 16, 16)]
                if mode == "PM":
                    wb1 = wv * (BETA if side == 0 else -BETA)
                    wb2 = (wv * wv) * (-BETA)
                else:
                    wb1 = None
                    wb2 = -wv if side == 0 else wv
                for d in range(D):
                    cd = cds[d]
                    e = plsc.load_gather(tbuf, (ridx, cd))
                    val = e * wb1 if wb1 is not None else e
                    if susi is not None:
                        a = (plsc.load_gather(tbuf, (Q + ridx, cd))
                             + plsc.load_gather(tbuf, (2 * Q + ridx, cd)))
                        val = val + a * wb2
                    plsc.store_scatter(tbuf, (3 * Q + ridx, cd), val)
                return 0

            lax.fori_loop(0, Q // 16, grp, 0)
            drow = pl.multiple_of(
                dst_off_by_side[side] + sid * RT + q * Q, 8)
            pltpu.sync_copy(tbuf.at[pl.ds(3 * Q, Q)], T.at[pl.ds(drow, Q)])
            return 0

        lax.fori_loop(0, RT // Q, qstep, 0)


def _edge_update(wid, eu, eio, z_in, z_out, T, bufs, ld, gt, st):
    first_pass = z_in is None

    def ebase(b):
        return pl.multiple_of(wid * EW + b * BLK, 128)

    def issue_load(b, s):
        iu, ii, tb = bufs[s]
        base = ebase(b)
        pltpu.async_copy(eu.at[pl.ds(base, BLK)], iu, ld[s])
        pltpu.async_copy(eio.at[pl.ds(base, BLK)], ii, ld[s])
        if not first_pass:
            pltpu.async_copy(z_in.at[pl.ds(base, BLK)], tb, ld[s])

    def wait_load(s):
        iu, ii, tb = bufs[s]
        pltpu.make_async_copy(eu.at[pl.ds(0, BLK)], iu, ld[s]).wait()
        pltpu.make_async_copy(eio.at[pl.ds(0, BLK)], ii, ld[s]).wait()
        if not first_pass:
            pltpu.make_async_copy(z_in.at[pl.ds(0, BLK)], tb, ld[s]).wait()

    def gather_side(s, idxr, add):
        _, _, tb = bufs[s]
        for c in range(NSUB):
            pltpu.async_copy(T.at[idxr.at[pl.ds(c * 128, 128)]],
                             tb.at[pl.ds(c * 128, 128)], gt[s], add=add)

    def wait_gathers(s, n):
        _, _, tb = bufs[s]
        for c in range(n):
            pltpu.make_async_copy(T.at[bufs[s][0].at[pl.ds(0, 128)]],
                                  tb.at[pl.ds((c % NSUB) * 128, 128)],
                                  gt[s]).wait()

    def compute(s):
        _, _, tb = bufs[s]

        def grp(j, _):
            ridx = j * 16 + _iota16()
            cds = _rot_cols()
            parts = [jnp.zeros((16,), F32) for _ in range(4)]
            vs = []
            for d in range(D):
                v = plsc.load_gather(tb, (ridx, cds[d]))
                vs.append(v)
                parts[d % 4] = parts[d % 4] + v * v
            sq = (parts[0] + parts[1]) + (parts[2] + parts[3])
            scale = jnp.minimum(
                1.0, LAMBDA2 * _fast_rsqrt(jnp.maximum(sq, 1e-24)))
            for d in range(D):
                plsc.store_scatter(tb, (ridx, cds[d]), vs[d] * scale)
            return 0

        lax.fori_loop(0, NG, grp, 0)

    def issue_store(b, s):
        _, _, tb = bufs[s]
        pltpu.async_copy(tb, z_out.at[pl.ds(ebase(b), BLK)], st[s])

    def wait_store(s):
        _, _, tb = bufs[s]
        pltpu.make_async_copy(tb, z_out.at[pl.ds(0, BLK)], st[s]).wait()

    def body(b, s, o, prefetch, first_waits):
        wait_load(s)
        if first_pass:
            gather_side(s, bufs[s][0], False)
            wait_gathers(s, NSUB)
            gather_side(s, bufs[s][1], True)
            wait_gathers(s, NSUB)
        else:
            gather_side(s, bufs[s][0], True)
            gather_side(s, bufs[s][1], True)
            wait_gathers(s, 2 * NSUB)
        if prefetch:
            if not first_waits:
                wait_store(o)
            issue_load(b + 1, o)
        compute(s)
        issue_store(b, s)

    issue_load(0, 0)
    body(0, 0, 1, True, True)
    body(1, 1, 0, True, False)

    def pair(p, _):
        b = 2 * p + 2
        body(b, 0, 1, True, False)
        body(b + 1, 1, 0, True, False)
        return 0

    lax.fori_loop(0, (NBLK - 4) // 2, pair, 0)
    body(NBLK - 2, 0, 1, True, False)
    body(NBLK - 1, 1, 0, False, False)
    wait_store(0)
    wait_store(1)


def _z1_body(eu, eio, emb_u, emb_i, w, z_out,
             iu0, ii0, tb0, iu1, ii1, tb1, wvb, T,
             ld0, ld1, gt0, gt1, st0, st1):
    sid = lax.axis_index("s")
    _build_tables(sid, emb_u, emb_i, w, None, tb0, wvb, T, "PM",
                  (0, 1), (0, NP))
    plsc.subcore_barrier()
    _edge_update(_wid(), eu, eio, None, z_out, T,
                 ((iu0, ii0, tb0), (iu1, ii1, tb1)),
                 (ld0, ld1), (gt0, gt1), (st0, st1))


def _zk_body(eu, eio, emb_u, emb_i, w, susi, z_in, z_out,
             iu0, ii0, tb0, iu1, ii1, tb1, wvb, T,
             ld0, ld1, gt0, gt1, st0, st1):
    sid = lax.axis_index("s")
    _build_tables(sid, emb_u, emb_i, w, susi, tb0, wvb, T, "PM",
                  (0, 1), (0, NP))
    plsc.subcore_barrier()
    _edge_update(_wid(), eu, eio, z_in, z_out, T,
                 ((iu0, ii0, tb0), (iu1, ii1, tb1)),
                 (ld0, ld1), (gt0, gt1), (st0, st1))


def _scatter_body(eu, eio, z_in, s_out,
                  iu20, ii20, zb0, iu21, ii21, zb1,
                  S, ld0, ld1, sc0, sc1):
    cid = lax.axis_index("c")
    sid = lax.axis_index("s")
    wid = cid * NS + sid
    z16 = jnp.zeros((16,), F32)
    bufs = ((iu20, ii20, zb0), (iu21, ii21, zb1))
    ld = (ld0, ld1)
    sc = (sc0, sc1)

    def zrow(r, _):
        plsc.store_scatter(zb0, (_splat(r), _iota16()), z16)
        plsc.store_scatter(zb0, (_splat(r), 16 + _iota16()), z16)
        return 0

    lax.fori_loop(0, BLK, zrow, 0)
    for half in range(2):
        trow = pl.multiple_of(half * NP + sid * RT, 8)
        for c0 in range(0, RT, BLK):
            n = min(BLK, RT - c0)
            pltpu.sync_copy(zb0.at[pl.ds(0, n)], S.at[pl.ds(trow + c0, n)])
    plsc.subcore_barrier()

    def issue_load(b, s):
        iu2, ii2, zb = bufs[s]
        base = pl.multiple_of(wid * EW + b * BLK, 128)
        pltpu.async_copy(z_in.at[pl.ds(base, BLK)], zb, ld[s])
        pltpu.async_copy(eu.at[pl.ds(base, BLK)], iu2, ld[s])
        pltpu.async_copy(eio.at[pl.ds(base, BLK)], ii2, ld[s])

    def wait_load(s):
        iu2, ii2, zb = bufs[s]
        pltpu.make_async_copy(z_in.at[pl.ds(0, BLK)], zb, ld[s]).wait()
        pltpu.make_async_copy(eu.at[pl.ds(0, BLK)], iu2, ld[s]).wait()
        pltpu.make_async_copy(eio.at[pl.ds(0, BLK)], ii2, ld[s]).wait()

    def issue_scatters(s):
        iu2, ii2, zb = bufs[s]
        for c in range(NSUB):
            srcv = zb.at[pl.ds(c * 128, 128)]
            pltpu.async_copy(srcv, S.at[iu2.at[pl.ds(c * 128, 128)]],
                             sc[s], add=True)
            pltpu.async_copy(srcv, S.at[ii2.at[pl.ds(c * 128, 128)]],
                             sc[s], add=True)

    def wait_scatters(s):
        iu2, ii2, zb = bufs[s]
        for c in range(NSUB):
            pltpu.make_async_copy(zb.at[pl.ds(c * 128, 128)],
                                  S.at[iu2.at[pl.ds(c * 128, 128)]],
                                  sc[s]).wait()
            pltpu.make_async_copy(zb.at[pl.ds(c * 128, 128)],
                                  S.at[ii2.at[pl.ds(c * 128, 128)]],
                                  sc[s]).wait()

    def body(b, s, o, prefetch, first):
        wait_load(s)
        issue_scatters(s)
        if prefetch:
            if not first:
                wait_scatters(o)
            issue_load(b + 1, o)

    issue_load(0, 0)
    body(0, 0, 1, True, True)
    body(1, 1, 0, True, False)

    def pair(p, _):
        b = 2 * p + 2
        body(b, 0, 1, True, False)
        body(b + 1, 1, 0, True, False)
        return 0

    lax.fori_loop(0, (NBLK - 4) // 2, pair, 0)
    body(NBLK - 2, 0, 1, True, False)
    body(NBLK - 1, 1, 0, False, False)
    wait_scatters(0)
    wait_scatters(1)

    plsc.subcore_barrier()
    for half in range(2):
        trow = pl.multiple_of(half * NP + sid * RT, 8)
        pltpu.sync_copy(S.at[pl.ds(trow, RT)],
                        s_out.at[cid, pl.ds(trow, RT)])


def _final_body(users, items, emb_u, emb_i, w, susi, gamma,
                ub, ibx, rows_u, rows_i, gbuf, tbuf, wvb, T):
    sid = lax.axis_index("s")
    wid = _wid()
    base = pl.multiple_of(wid * PB, 8)
    pltpu.sync_copy(users.at[pl.ds(base, PB)], ub)
    pltpu.sync_copy(items.at[pl.ds(base, PB)], ibx)
    for side, idxr, rows in ((0, ub, rows_u), (1, ibx, rows_i)):
        _build_tables(sid, emb_u, emb_i, w, susi, tbuf, wvb, T, "X",
                      (side,), (0, 0))
        plsc.subcore_barrier()
        for s in range(PB // 128):
            pltpu.sync_copy(T.at[idxr.at[pl.ds(s * 128, 128)]],
                            rows.at[pl.ds(s * 128, 128)])
        plsc.subcore_barrier()

    def grp(j, _):
        ridx = j * 16 + _iota16()
        cds = _rot_cols()
        parts = [jnp.zeros((16,), F32) for _ in range(4)]
        for d in range(D):
            cd = cds[d]
            parts[d % 4] = parts[d % 4] + (
                plsc.load_gather(rows_u, (ridx, cd))
                * plsc.load_gather(rows_i, (ridx, cd)))
        gbuf[pl.ds(j * 16, 16)] = (parts[0] + parts[1]) + (parts[2]
                                                           + parts[3])
        return 0

    lax.fori_loop(0, PB // 16, grp, 0)
    pltpu.sync_copy(gbuf, gamma.at[pl.ds(base, PB)])


def _mk(body, out_type, scratch):
    return pl.kernel(
        body, out_type=out_type, mesh=_MESH, scratch_types=scratch,
        compiler_params=pltpu.CompilerParams(
            needs_layout_passes=False, use_tc_tiling_on_sc=False))


def kernel(users, items, user_emb, item_emb, edge_user, edge_item):
    pad_e = jnp.full((NNZP - NNZ,), NU, I32)
    eu = jnp.concatenate([edge_user.astype(I32), pad_e])
    ei = jnp.concatenate([edge_item.astype(I32), pad_e])
    eio = ei + NP
    emb_u = jnp.pad(user_emb, ((0, NP - NU), (0, 0)))
    emb_i = jnp.pad(item_emb, ((0, NP - NI), (0, 0)))

    k_deg = _mk(_deg_body, jax.ShapeDtypeStruct((2 * NP,), F32), [
        pltpu.VMEM((NP,), F32),
        pltpu.VMEM((NP,), F32),
        pltpu.VMEM((B_T1,), I32),
        pltpu.VMEM((B_T1,), I32),
        pltpu.VMEM((RT,), F32),
        pltpu.VMEM_SHARED((NS * NP,), F32),
    ])
    w = k_deg(eu, ei)

    sems6 = [pltpu.SemaphoreType.DMA] * 6
    sems4 = [pltpu.SemaphoreType.DMA] * 4
    slot = [
        pltpu.VMEM((BLK,), I32),
        pltpu.VMEM((BLK,), I32),
        pltpu.VMEM((BLK, D), F32),
    ]
    zpass_scratch = (slot + slot + [
        pltpu.VMEM((Q,), F32),
        pltpu.VMEM_SHARED((2 * NP, D), F32),
    ] + sems6)
    zshape = jax.ShapeDtypeStruct((NNZP, D), F32)
    sshape = jax.ShapeDtypeStruct((NC, 2 * NP, D), F32)
    scat_scratch = (slot + slot + [
        pltpu.VMEM_SHARED((2 * NP, D), F32),
    ] + sems4)

    k_z1 = _mk(_z1_body, zshape, zpass_scratch)
    k_zk = _mk(_zk_body, zshape, zpass_scratch)
    k_sc = _mk(_scatter_body, sshape, scat_scratch)
    k_fin = _mk(_final_body, jax.ShapeDtypeStruct((BATCH,), F32), [
        pltpu.VMEM((PB,), I32),
        pltpu.VMEM((PB,), I32),
        pltpu.VMEM((PB, D), F32),
        pltpu.VMEM((PB, D), F32),
        pltpu.VMEM((PB,), F32),
        pltpu.VMEM((4 * Q, D), F32),
        pltpu.VMEM((Q,), F32),
        pltpu.VMEM_SHARED((NP, D), F32),
    ])

    z = k_z1(eu, eio, emb_u, emb_i, w)
    for _ in range(K_LAYERS - 1):
        s_part = k_sc(eu, eio, z)
        z = k_zk(eu, eio, emb_u, emb_i, w, s_part, z)
    s_part = k_sc(eu, eio, z)

    gamma = k_fin(users.astype(I32), items.astype(I32), emb_u, emb_i,
                  w, s_part)
    return gamma
